# baseline (speedup 1.0000x reference)
import functools

import jax
import jax.numpy as jnp
from jax import lax
from jax.experimental import pallas as pl
from jax.experimental.pallas import tpu as pltpu
from jax.experimental.pallas import tpu_sc as plsc

B, T = 16384, 200
V, D = 32, 8
M = B * T
NC, NS, L = 2, 16, 16
NW = NC * NS
GSZ = 2048
NG = M // GSZ
GPW = NG // NW
GPT = B // GSZ

IDX_HBM_SHAPE = (25 * (B // 128), 8, 128)


def _vperm(src, idx):
    return lax.gather(
        src,
        idx[:, None],
        dimension_numbers=lax.GatherDimensionNumbers(
            offset_dims=(), collapsed_slice_dims=(0,), start_index_map=(0,)
        ),
        slice_sizes=(1,),
        mode=lax.GatherScatterMode.PROMISE_IN_BOUNDS,
    )


@functools.partial(
    pl.kernel,
    mesh=plsc.VectorSubcoreMesh(core_axis_name="c", subcore_axis_name="s"),
    out_type=jax.ShapeDtypeStruct((M * D,), jnp.float32),
    compiler_params=pltpu.CompilerParams(needs_layout_passes=False),
    scratch_types=[
        pltpu.VMEM((V * D,), jnp.float32),
        pltpu.VMEM((GSZ // 128, 128), jnp.int32),
        pltpu.VMEM((GSZ // 128, 128), jnp.int32),
        pltpu.VMEM((GSZ * D,), jnp.float32),
        pltpu.VMEM((GSZ * D,), jnp.float32),
        pltpu.SemaphoreType.DMA,
        pltpu.SemaphoreType.DMA,
        pltpu.SemaphoreType.DMA,
        pltpu.SemaphoreType.DMA,
    ],
)
def _gather_kernel(idx_hbm, wt_hbm, out_hbm, tabt_v, idx_v0, idx_v1,
                   out_v0, out_v1, isem0, isem1, osem0, osem1):
    wid = lax.axis_index("s") * NC + lax.axis_index("c")
    g0 = wid * GPW
    pltpu.sync_copy(wt_hbm, tabt_v)

    idx_bufs = (idx_v0, idx_v1)
    out_bufs = (out_v0, out_v1)
    isems = (isem0, isem1)
    osems = (osem0, osem1)

    def in_copy(gi, sel):
        t = gi // GPT
        btg = gi % GPT
        row0 = (t // 8) * (B // 128) + btg * (GSZ // 128)
        src = idx_hbm.at[pl.ds(row0, GSZ // 128), t % 8, :]
        return pltpu.make_async_copy(src, idx_bufs[sel], isems[sel])

    def out_copy(gi, sel):
        t = gi // GPT
        btg = gi % GPT
        dst = out_hbm.at[pl.ds(t * (B * D) + btg * (GSZ * D), GSZ * D)]
        return pltpu.make_async_copy(out_bufs[sel], dst, osems[sel])

    def compute(sel):
        idx_v = idx_bufs[sel]
        out_v = out_bufs[sel]
        tjs = [
            (tabt_v[pl.ds(j * V, L)], tabt_v[pl.ds(j * V + L, L)])
            for j in range(D)
        ]

        @plsc.parallel_loop(0, GSZ // 128, 1, unroll=1)
        def blk(k8):
            base2 = k8 << 10
            for kk in range(8):
                iv = idx_v[k8, pl.ds(kk * L, L)]
                m = iv < L
                ivlo = jnp.bitwise_and(iv, L - 1)
                for j in range(D):
                    lo, hi = tjs[j]
                    val = jnp.where(m, _vperm(lo, ivlo), _vperm(hi, ivlo))
                    out_v[pl.ds(base2 + (j * 128 + kk * L), L)] = val

    in_copy(g0, 0).start()

    def pair_body(p, carry):
        i0 = g0 + 2 * p
        i1 = i0 + 1
        in_copy(i1, 1).start()
        in_copy(i0, 0).wait()

        @pl.when(p > 0)
        def _():
            out_copy(i0 - 2, 0).wait()

        compute(0)
        out_copy(i0, 0).start()

        @pl.when(p < (GPW - 1) // 2)
        def _():
            in_copy(i0 + 2, 0).start()

        in_copy(i1, 1).wait()

        @pl.when(p > 0)
        def _():
            out_copy(i1 - 2, 1).wait()

        compute(1)
        out_copy(i1, 1).start()
        return carry

    lax.fori_loop(0, GPW // 2, pair_body, 0)
    if GPW % 2:
        gl = g0 + GPW - 1
        in_copy(gl, 0).wait()
        out_copy(gl - 2, 0).wait()
        compute(0)
        out_copy(gl, 0).start()
        out_copy(gl - 1, 1).wait()
        out_copy(gl, 0).wait()
    else:
        out_copy(g0 + GPW - 2, 0).wait()
        out_copy(g0 + GPW - 1, 1).wait()


def kernel(input, W):
    idx_t = (
        input.T.reshape(25, 8, B // 128, 128)
        .transpose(0, 2, 1, 3)
        .reshape(IDX_HBM_SHAPE)
    )
    out_flat = _gather_kernel(idx_t, W.T.reshape(-1))
    return (
        out_flat.reshape(T, B // 128, D, 128)
        .transpose(1, 3, 0, 2)
        .reshape(B, T, D)
    )


# speedup vs baseline: 1.0575x; 1.0575x over previous
import functools

import jax
import jax.numpy as jnp
from jax import lax
from jax.experimental import pallas as pl
from jax.experimental.pallas import tpu as pltpu
from jax.experimental.pallas import tpu_sc as plsc

B, T = 16384, 200
V, D = 32, 8
M = B * T
NC, NS, L = 2, 16, 16
NW = NC * NS
GSZ = 4096
NG = M // GSZ
GPW = NG // NW
GPT = B // GSZ

IDX_HBM_SHAPE = (25 * (B // 128), 8, 128)


def _vperm(src, idx):
    return lax.gather(
        src,
        idx[:, None],
        dimension_numbers=lax.GatherDimensionNumbers(
            offset_dims=(), collapsed_slice_dims=(0,), start_index_map=(0,)
        ),
        slice_sizes=(1,),
        mode=lax.GatherScatterMode.PROMISE_IN_BOUNDS,
    )


@functools.partial(
    pl.kernel,
    mesh=plsc.VectorSubcoreMesh(core_axis_name="c", subcore_axis_name="s"),
    out_type=jax.ShapeDtypeStruct((M * D,), jnp.float32),
    compiler_params=pltpu.CompilerParams(needs_layout_passes=False),
    scratch_types=[
        pltpu.VMEM((V * D,), jnp.float32),
        pltpu.VMEM((GSZ // 128, 128), jnp.int32),
        pltpu.VMEM((GSZ // 128, 128), jnp.int32),
        pltpu.VMEM((GSZ * D,), jnp.float32),
        pltpu.VMEM((GSZ * D,), jnp.float32),
        pltpu.SemaphoreType.DMA,
        pltpu.SemaphoreType.DMA,
        pltpu.SemaphoreType.DMA,
        pltpu.SemaphoreType.DMA,
    ],
)
def _gather_kernel(idx_hbm, wt_hbm, out_hbm, tabt_v, idx_v0, idx_v1,
                   out_v0, out_v1, isem0, isem1, osem0, osem1):
    wid = lax.axis_index("s") * NC + lax.axis_index("c")
    g0 = wid * GPW
    pltpu.sync_copy(wt_hbm, tabt_v)

    idx_bufs = (idx_v0, idx_v1)
    out_bufs = (out_v0, out_v1)
    isems = (isem0, isem1)
    osems = (osem0, osem1)

    def in_copy(gi, sel):
        t = gi // GPT
        btg = gi % GPT
        row0 = (t // 8) * (B // 128) + btg * (GSZ // 128)
        src = idx_hbm.at[pl.ds(row0, GSZ // 128), t % 8, :]
        return pltpu.make_async_copy(src, idx_bufs[sel], isems[sel])

    def out_copy(gi, sel):
        t = gi // GPT
        btg = gi % GPT
        dst = out_hbm.at[pl.ds(t * (B * D) + btg * (GSZ * D), GSZ * D)]
        return pltpu.make_async_copy(out_bufs[sel], dst, osems[sel])

    def compute(sel):
        idx_v = idx_bufs[sel]
        out_v = out_bufs[sel]
        tjs = [
            (tabt_v[pl.ds(j * V, L)], tabt_v[pl.ds(j * V + L, L)])
            for j in range(D)
        ]

        @plsc.parallel_loop(0, GSZ // 128, 1, unroll=1)
        def blk(k8):
            base2 = k8 << 10
            for kk in range(8):
                iv = idx_v[k8, pl.ds(kk * L, L)]
                m = iv < L
                ivlo = jnp.bitwise_and(iv, L - 1)
                for j in range(D):
                    lo, hi = tjs[j]
                    val = jnp.where(m, _vperm(lo, ivlo), _vperm(hi, ivlo))
                    out_v[pl.ds(base2 + (j * 128 + kk * L), L)] = val

    in_copy(g0, 0).start()

    def pair_body(p, carry):
        i0 = g0 + 2 * p
        i1 = i0 + 1
        in_copy(i1, 1).start()
        in_copy(i0, 0).wait()

        @pl.when(p > 0)
        def _():
            out_copy(i0 - 2, 0).wait()

        compute(0)
        out_copy(i0, 0).start()

        @pl.when(p < (GPW - 1) // 2)
        def _():
            in_copy(i0 + 2, 0).start()

        in_copy(i1, 1).wait()

        @pl.when(p > 0)
        def _():
            out_copy(i1 - 2, 1).wait()

        compute(1)
        out_copy(i1, 1).start()
        return carry

    lax.fori_loop(0, GPW // 2, pair_body, 0)
    if GPW % 2:
        gl = g0 + GPW - 1
        in_copy(gl, 0).wait()
        out_copy(gl - 2, 0).wait()
        compute(0)
        out_copy(gl, 0).start()
        out_copy(gl - 1, 1).wait()
        out_copy(gl, 0).wait()
    else:
        out_copy(g0 + GPW - 2, 0).wait()
        out_copy(g0 + GPW - 1, 1).wait()


def kernel(input, W):
    idx_t = (
        input.T.reshape(25, 8, B // 128, 128)
        .transpose(0, 2, 1, 3)
        .reshape(IDX_HBM_SHAPE)
    )
    out_flat = _gather_kernel(idx_t, W.T.reshape(-1))
    return (
        out_flat.reshape(T, B // 128, D, 128)
        .transpose(1, 3, 0, 2)
        .reshape(B, T, D)
    )
